# baseline (speedup 1.0000x reference)
import jax
import jax.numpy as jnp
from jax.experimental import pallas as pl
from jax.experimental.pallas import tpu as pltpu

N = 74
EMB = 64
G = 128
H = 8
ER = 64
K = 5

_F32 = jnp.float32
_BIG = 3e38


def _body(dist_h, markov_h, mask_h, demand_h, wkday_h, emb_h,
          srw_h, srb_h, srootw_h,
          gatw_h, asrc_h, adst_h, aedge_h, gatb_h,
          ew_h, eb_h, c1w_h, c1b_h, c2w_h, c2b_h, out_ref,
          dist_ref, markov_ref, mask_ref, demand_ref, wkday_ref, emb_ref,
          srw_ref, srb_ref, srootw_ref,
          gatw_ref, asrc_ref, adst_ref, aedge_ref, gatb_ref,
          ew_ref, eb_ref, c1w_ref, c1b_ref, c2w_ref, c2b_ref,
          s0, s1, s2, s3, s4, s5):
    cps0 = [pltpu.make_async_copy(demand_h, demand_ref, s0),
            pltpu.make_async_copy(emb_h, emb_ref, s0)]
    cps1 = [pltpu.make_async_copy(mask_h, mask_ref, s1),
            pltpu.make_async_copy(srw_h, srw_ref, s1),
            pltpu.make_async_copy(srb_h, srb_ref, s1),
            pltpu.make_async_copy(srootw_h, srootw_ref, s1)]
    cps2 = [pltpu.make_async_copy(gatw_h, gatw_ref, s2),
            pltpu.make_async_copy(markov_h, markov_ref, s2),
            pltpu.make_async_copy(asrc_h, asrc_ref, s2),
            pltpu.make_async_copy(adst_h, adst_ref, s2),
            pltpu.make_async_copy(aedge_h, aedge_ref, s2),
            pltpu.make_async_copy(gatb_h, gatb_ref, s2)]
    cps3 = [pltpu.make_async_copy(ew_h, ew_ref, s3),
            pltpu.make_async_copy(eb_h, eb_ref, s3)]
    cps4 = [pltpu.make_async_copy(c1w_h, c1w_ref, s4)]
    cps5 = [pltpu.make_async_copy(dist_h, dist_ref, s5),
            pltpu.make_async_copy(wkday_h, wkday_ref, s5),
            pltpu.make_async_copy(c1b_h, c1b_ref, s5),
            pltpu.make_async_copy(c2w_h, c2w_ref, s5),
            pltpu.make_async_copy(c2b_h, c2b_ref, s5)]
    for cp in cps0 + cps1 + cps2 + cps3 + cps4 + cps5:
        cp.start()

    colids = jax.lax.broadcasted_iota(jnp.int32, (N, N), 1)
    rowids = jax.lax.broadcasted_iota(jnp.int32, (N, N), 0)

    for cp in cps0:
        cp.wait()
    dem = demand_ref[...]
    oh = (dem == colids).astype(_F32)
    x0 = jnp.dot(oh, emb_ref[...], preferred_element_type=_F32)

    for cp in cps1:
        cp.wait()
    maskv = mask_ref[...]
    deg = jnp.maximum(jnp.sum(maskv, axis=1, keepdims=True), 1.0)
    agg = jnp.dot(maskv, x0, preferred_element_type=_F32) / deg
    x1 = (jnp.dot(agg, srw_ref[...], preferred_element_type=_F32)
          + srb_ref[...]
          + jnp.dot(x0, srootw_ref[...], preferred_element_type=_F32))

    a = x1
    chunks = []
    for i0 in range(0, N, 8):
        sz = min(8, N - i0)
        ach = a[i0:i0 + sz]
        diff = ach[:, None, :] - a[None, :, :]
        chunks.append(jnp.sum(diff * diff, axis=2))
    d2 = jnp.concatenate(chunks, axis=0)
    d2 = jnp.where(rowids == colids, d2 + 1e9, d2)

    colf = colids.astype(_F32)
    M = jnp.zeros((N, N), _F32)
    d2w = d2
    for _ in range(K):
        mval = jnp.min(d2w, axis=1, keepdims=True)
        ismin = d2w == mval
        sel = jnp.min(jnp.where(ismin, colf, _BIG), axis=1, keepdims=True)
        onehot = colf == sel
        M = jnp.where(onehot, 1.0, M)
        d2w = jnp.where(onehot, _BIG, d2w)
    Mb = M > 0.0

    xr = jnp.maximum(x1, 0.0)
    for cp in cps2:
        cp.wait()
    h2 = jnp.dot(xr, gatw_ref[...], preferred_element_type=_F32)
    markov = markov_ref[...]
    cn = (((0,), (0,)), ((), ()))
    cl = (((1,), (1,)), ((), ()))
    head_outs = []
    for hd in range(H):
        hsl = h2[:, hd * G:(hd + 1) * G]
        asr = jax.lax.dot_general(hsl, asrc_ref[hd:hd + 1, :], cl,
                                  preferred_element_type=_F32)
        ads = jax.lax.dot_general(adst_ref[hd:hd + 1, :], hsl, cl,
                                  preferred_element_type=_F32)
        ae = aedge_ref[0:1, hd:hd + 1]
        e = asr + ads + markov * ae
        e = jnp.where(e >= 0.0, e, 0.2 * e)
        m = jnp.max(jnp.where(Mb, e, -_BIG), axis=0, keepdims=True)
        m = jnp.where(m < -1e37, 0.0, m)
        ex = jnp.exp(e - m) * M
        s = jnp.sum(ex, axis=0, keepdims=True)
        denom = jnp.where(s == 0.0, 1.0, s)
        alpha = ex / denom
        head_outs.append(jax.lax.dot_general(alpha, hsl, cn,
                                             preferred_element_type=_F32))
    x2 = jnp.concatenate(head_outs, axis=1) + gatb_ref[...]
    x2 = jnp.maximum(x2, 0.0)

    for cp in cps3:
        cp.wait()
    A = jnp.dot(x2, ew_ref[0:H * G], preferred_element_type=_F32)
    B = jnp.dot(x2, ew_ref[H * G:2 * H * G], preferred_element_type=_F32)
    eb = eb_ref[...]
    A2 = jnp.concatenate([A + eb, A + eb], axis=1)
    blocks = []
    for c in range(N // 2):
        Bpair = jnp.concatenate([B[2 * c:2 * c + 1], B[2 * c + 1:2 * c + 2]],
                                axis=1)
        blocks.append(jnp.maximum(A2 + Bpair, 0.0))
    R = jnp.concatenate(blocks, axis=1)
    for cp in cps4:
        cp.wait()
    acc = jnp.dot(R, c1w_ref[0:N * ER], preferred_element_type=_F32)

    for cp in cps5:
        cp.wait()
    dist = dist_ref[...]
    dmin = jnp.min(dist)
    dmax = jnp.max(dist)
    dn = (dist - dmin) / (dmax - dmin)
    wk7 = (jax.lax.broadcasted_iota(jnp.int32, (N, 7), 1)
           == wkday_ref[0, 0]).astype(_F32)
    Z = jnp.concatenate([dn, markov, wk7], axis=1)
    acc = acc + jnp.dot(Z, c1w_ref[N * ER:N * ER + 2 * N + 7],
                        preferred_element_type=_F32)
    out1 = jnp.maximum(acc + c1b_ref[...], 0.0)
    out_ref[...] = (jnp.dot(out1, c2w_ref[...], preferred_element_type=_F32)
                    + c2b_ref[...])


def kernel(dist, stops, weekday, vehicles, markov, demand, capacity, mask,
           emb, sage_rel_w, sage_rel_b, sage_root_w,
           gat_w, gat_att_src, gat_att_dst, gat_att_edge, gat_b,
           edge_w, edge_b, comb1_w, comb1_b, comb2_w, comb2_b):
    del stops, vehicles, capacity
    f32 = _F32
    demand2d = demand.astype(jnp.int32).reshape(N, 1)
    wkday2d = jnp.asarray(weekday, jnp.int32).reshape(1, 1)
    args = (
        dist, markov, mask, demand2d, wkday2d, emb,
        sage_rel_w, sage_rel_b.reshape(1, G), sage_root_w,
        gat_w, gat_att_src, gat_att_dst, gat_att_edge.reshape(1, H),
        gat_b.reshape(1, H * G),
        edge_w, edge_b.reshape(1, ER),
        comb1_w, comb1_b.reshape(1, N),
        comb2_w, comb2_b.reshape(1, N),
    )
    hbm = pl.BlockSpec(memory_space=pltpu.MemorySpace.HBM)
    return pl.pallas_call(
        _body,
        out_shape=jax.ShapeDtypeStruct((N, N), f32),
        in_specs=[hbm] * 20,
        scratch_shapes=[
            pltpu.VMEM((N, N), f32),
            pltpu.VMEM((N, N), f32),
            pltpu.VMEM((N, N), f32),
            pltpu.VMEM((N, 1), jnp.int32),
            pltpu.VMEM((1, 1), jnp.int32),
            pltpu.VMEM((N, EMB), f32),
            pltpu.VMEM((EMB, G), f32),
            pltpu.VMEM((1, G), f32),
            pltpu.VMEM((EMB, G), f32),
            pltpu.VMEM((G, H * G), f32),
            pltpu.VMEM((H, G), f32),
            pltpu.VMEM((H, G), f32),
            pltpu.VMEM((1, H), f32),
            pltpu.VMEM((1, H * G), f32),
            pltpu.VMEM((2 * H * G, ER), f32),
            pltpu.VMEM((1, ER), f32),
            pltpu.VMEM((N * ER + 2 * N + 7, N), f32),
            pltpu.VMEM((1, N), f32),
            pltpu.VMEM((N, N), f32),
            pltpu.VMEM((1, N), f32),
            pltpu.SemaphoreType.DMA,
            pltpu.SemaphoreType.DMA,
            pltpu.SemaphoreType.DMA,
            pltpu.SemaphoreType.DMA,
            pltpu.SemaphoreType.DMA,
            pltpu.SemaphoreType.DMA,
        ],
    )(*args)


# speedup vs baseline: 1.1007x; 1.1007x over previous
import jax
import jax.numpy as jnp
from jax.experimental import pallas as pl
from jax.experimental.pallas import tpu as pltpu

N = 74
EMB = 64
G = 128
H = 8
ER = 64
K = 5

_F32 = jnp.float32
_BIG = 3e38


def _body(dist_ref, markov_ref, mask_ref, demand_ref, wkday_ref, emb_ref,
          srw_ref, srb_ref, srootw_ref,
          gatw_hbm, asrc_ref, adst_ref, aedge_ref, gatb_ref,
          ew_hbm, eb_ref,
          c1w_hbm, c1b_ref,
          c2w_ref, c2b_ref, out_ref,
          gatw_ref, ew_ref, c1w_ref, sem_g, sem_e, sem_c):
    cp_g = pltpu.make_async_copy(gatw_hbm, gatw_ref, sem_g)
    cp_e = pltpu.make_async_copy(ew_hbm, ew_ref, sem_e)
    cp_c = pltpu.make_async_copy(c1w_hbm, c1w_ref, sem_c)
    cp_g.start()
    cp_e.start()
    cp_c.start()

    colids = jax.lax.broadcasted_iota(jnp.int32, (N, N), 1)
    rowids = jax.lax.broadcasted_iota(jnp.int32, (N, N), 0)

    dist = dist_ref[...]
    dmin = jnp.min(dist)
    dmax = jnp.max(dist)
    dn = (dist - dmin) / (dmax - dmin)

    dem = demand_ref[...]
    oh = (dem == colids).astype(_F32)
    x0 = jnp.dot(oh, emb_ref[...], preferred_element_type=_F32)

    maskv = mask_ref[...]
    deg = jnp.maximum(jnp.sum(maskv, axis=1, keepdims=True), 1.0)
    agg = jnp.dot(maskv, x0, preferred_element_type=_F32) / deg
    x1 = (jnp.dot(agg, srw_ref[...], preferred_element_type=_F32)
          + srb_ref[...]
          + jnp.dot(x0, srootw_ref[...], preferred_element_type=_F32))

    xr = jnp.maximum(x1, 0.0)
    cp_g.wait()
    h2 = jnp.dot(xr, gatw_ref[...], preferred_element_type=_F32)
    cl = (((1,), (1,)), ((), ()))
    asr_l = []
    ads_l = []
    for hd in range(H):
        hsl = h2[:, hd * G:(hd + 1) * G]
        asr_l.append(jax.lax.dot_general(hsl, asrc_ref[hd:hd + 1, :], cl,
                                         preferred_element_type=_F32))
        ads_l.append(jax.lax.dot_general(adst_ref[hd:hd + 1, :], hsl, cl,
                                         preferred_element_type=_F32))

    a = x1
    chunks = []
    for i0 in range(0, N, 8):
        sz = min(8, N - i0)
        ach = a[i0:i0 + sz]
        diff = ach[:, None, :] - a[None, :, :]
        chunks.append(jnp.sum(diff * diff, axis=2))
    d2 = jnp.concatenate(chunks, axis=0)
    d2 = jnp.where(rowids == colids, d2 + 1e9, d2)

    colf = colids.astype(_F32)
    M = jnp.zeros((N, N), _F32)
    d2w = d2
    for _ in range(K):
        mval = jnp.min(d2w, axis=1, keepdims=True)
        ismin = d2w == mval
        sel = jnp.min(jnp.where(ismin, colf, _BIG), axis=1, keepdims=True)
        onehot = colf == sel
        M = jnp.where(onehot, 1.0, M)
        d2w = jnp.where(onehot, _BIG, d2w)
    Mb = M > 0.0

    markov = markov_ref[...]
    cn = (((0,), (0,)), ((), ()))
    head_outs = []
    for hd in range(H):
        hsl = h2[:, hd * G:(hd + 1) * G]
        ae = aedge_ref[0:1, hd:hd + 1]
        e = asr_l[hd] + ads_l[hd] + markov * ae
        e = jnp.where(e >= 0.0, e, 0.2 * e)
        m = jnp.max(jnp.where(Mb, e, -_BIG), axis=0, keepdims=True)
        m = jnp.where(m < -1e37, 0.0, m)
        ex = jnp.exp(e - m) * M
        s = jnp.sum(ex, axis=0, keepdims=True)
        denom = jnp.where(s == 0.0, 1.0, s)
        alpha = ex / denom
        head_outs.append(jax.lax.dot_general(alpha, hsl, cn,
                                             preferred_element_type=_F32))
    x2 = jnp.concatenate(head_outs, axis=1) + gatb_ref[...]
    x2 = jnp.maximum(x2, 0.0)

    cp_e.wait()
    A = jnp.dot(x2, ew_ref[0:H * G], preferred_element_type=_F32)
    B = jnp.dot(x2, ew_ref[H * G:2 * H * G], preferred_element_type=_F32)
    eb = eb_ref[...]
    A2 = jnp.concatenate([A + eb, A + eb], axis=1)
    blocks = []
    for c in range(N // 2):
        Bpair = jnp.concatenate([B[2 * c:2 * c + 1], B[2 * c + 1:2 * c + 2]],
                                axis=1)
        blocks.append(jnp.maximum(A2 + Bpair, 0.0))
    R = jnp.concatenate(blocks, axis=1)
    cp_c.wait()
    acc = jnp.dot(R, c1w_ref[0:N * ER], preferred_element_type=_F32)

    wk7 = (jax.lax.broadcasted_iota(jnp.int32, (N, 7), 1)
           == wkday_ref[0, 0]).astype(_F32)
    Z = jnp.concatenate([dn, markov, wk7], axis=1)
    acc = acc + jnp.dot(Z, c1w_ref[N * ER:N * ER + 2 * N + 7],
                        preferred_element_type=_F32)
    out1 = jnp.maximum(acc + c1b_ref[...], 0.0)
    out_ref[...] = (jnp.dot(out1, c2w_ref[...], preferred_element_type=_F32)
                    + c2b_ref[...])


def kernel(dist, stops, weekday, vehicles, markov, demand, capacity, mask,
           emb, sage_rel_w, sage_rel_b, sage_root_w,
           gat_w, gat_att_src, gat_att_dst, gat_att_edge, gat_b,
           edge_w, edge_b, comb1_w, comb1_b, comb2_w, comb2_b):
    del stops, vehicles, capacity
    f32 = _F32
    demand2d = demand.astype(jnp.int32).reshape(N, 1)
    wkday2d = jnp.asarray(weekday, jnp.int32).reshape(1, 1)
    args = (
        dist, markov, mask, demand2d, wkday2d, emb,
        sage_rel_w, sage_rel_b.reshape(1, G), sage_root_w,
        gat_w, gat_att_src, gat_att_dst, gat_att_edge.reshape(1, H),
        gat_b.reshape(1, H * G),
        edge_w, edge_b.reshape(1, ER),
        comb1_w, comb1_b.reshape(1, N),
        comb2_w, comb2_b.reshape(1, N),
    )
    vm = pl.BlockSpec(memory_space=pltpu.MemorySpace.VMEM)
    hbm = pl.BlockSpec(memory_space=pltpu.MemorySpace.HBM)
    specs = [vm] * 9 + [hbm] + [vm] * 4 + [hbm, vm, hbm, vm, vm, vm]
    return pl.pallas_call(
        _body,
        out_shape=jax.ShapeDtypeStruct((N, N), f32),
        in_specs=specs,
        scratch_shapes=[
            pltpu.VMEM((G, H * G), f32),
            pltpu.VMEM((2 * H * G, ER), f32),
            pltpu.VMEM((N * ER + 2 * N + 7, N), f32),
            pltpu.SemaphoreType.DMA,
            pltpu.SemaphoreType.DMA,
            pltpu.SemaphoreType.DMA,
        ],
    )(*args)
